# baseline (device time: 264018 ns/iter reference)
import jax
import jax.numpy as jnp
from jax import lax
from jax.experimental import pallas as pl
from jax.experimental.pallas import tpu as pltpu

N_Y = 4
N_STEP = N_Y - 1

S_R = 0
S_L = 1
S_XR = 2
S_XL = 3


def kernel(Q, K, V):
    b, s, h, d = Q.shape
    bh = b * h
    hh = bh // 2
    scale = d ** -0.5

    Qt = Q.transpose(0, 2, 3, 1).reshape(bh, d, s) * scale
    Kt = K.transpose(0, 2, 3, 1).reshape(2, hh, d, s)
    Vt = V.transpose(0, 2, 3, 1).reshape(2, hh, d, s)
    KVt = jnp.stack([Kt, Vt], axis=1)

    def body(q_ref, kv_ref, out_ref, kvbuf, m_buf, l_buf, send_sems, recv_sems):
        my_x = lax.axis_index("x")
        my_y = lax.axis_index("y")
        my_z = lax.axis_index("z")
        mh = my_x
        oh = 1 - my_x
        twin_dev = (1 - my_x, my_y, my_z)
        here = (my_x, my_y, my_z)
        has_left = my_y > 0
        has_right = my_y < N_Y - 1

        left_dev = (my_x, jnp.maximum(my_y - 1, 0), my_z)
        right_dev = (my_x, jnp.minimum(my_y + 1, N_Y - 1), my_z)

        barrier = pltpu.get_barrier_semaphore()
        for dev in (twin_dev, left_dev, right_dev):
            pl.semaphore_signal(
                barrier, inc=1, device_id=dev,
                device_id_type=pl.DeviceIdType.MESH,
            )
        pl.semaphore_wait(barrier, 3)

        def mk(stream, t, src, dst, dev):
            return pltpu.make_async_remote_copy(
                src_ref=src,
                dst_ref=dst,
                send_sem=send_sems.at[stream, t],
                recv_sem=recv_sems.at[stream, t],
                device_id=dev,
                device_id_type=pl.DeviceIdType.MESH,
            )

        def consume_own():
            def hb(i, carry):
                q = q_ref[i]
                ih, ir = i // hh, i % hh
                S0 = lax.dot_general(
                    q, kv_ref[ih, 0, ir], (((0,), (0,)), ((), ())),
                    preferred_element_type=jnp.float32)
                m0 = jnp.max(S0, axis=1)
                P = jnp.exp(S0 - m0[:, None])
                out_ref[i] = lax.dot_general(
                    kv_ref[ih, 1, ir], P, (((1,), (1,)), ((), ())),
                    preferred_element_type=jnp.float32)
                m_buf[i] = m0
                l_buf[i] = jnp.sum(P, axis=1)
                return carry
            lax.fori_loop(0, bh, hb, 0)

        def consume_slot(sl):
            def hb(i, carry):
                q = q_ref[i]
                ih, ir = i // hh, i % hh
                Sc = lax.dot_general(
                    q, kvbuf[sl, ih, 0, ir], (((0,), (0,)), ((), ())),
                    preferred_element_type=jnp.float32)
                m_old = m_buf[i]
                m_c = jnp.max(Sc, axis=1)
                m_new = jnp.maximum(m_old, m_c)
                alpha = jnp.exp(m_old - m_new)
                P = jnp.exp(Sc - m_new[:, None])
                pv = lax.dot_general(
                    kvbuf[sl, ih, 1, ir], P, (((1,), (1,)), ((), ())),
                    preferred_element_type=jnp.float32)
                out_ref[i] = out_ref[i] * alpha[None, :] + pv
                m_buf[i] = m_new
                l_buf[i] = l_buf[i] * alpha + jnp.sum(P, axis=1)
                return carry
            lax.fori_loop(0, bh, hb, 0)

        @pl.when(has_right)
        def _():
            mk(S_R, 0, kv_ref.at[mh], kvbuf.at[my_y, mh], right_dev).start()

        @pl.when(has_left)
        def _():
            mk(S_L, 0, kv_ref.at[mh], kvbuf.at[my_y - 1, mh],
               left_dev).start()

        consume_own()

        for t in range(N_STEP):
            @pl.when(my_y - 1 - t >= 0)
            def _(t=t):
                c = my_y - 1 - t
                mk(S_R, t, kvbuf.at[c, mh], kvbuf.at[c, mh],
                   here).wait_recv()

            @pl.when(my_y - 1 - t >= 0)
            def _(t=t):
                c = my_y - 1 - t
                mk(S_XR, t, kvbuf.at[c, mh], kvbuf.at[c, mh],
                   twin_dev).start()

            @pl.when(my_y + 1 + t <= N_Y - 1)
            def _(t=t):
                c = my_y + 1 + t
                mk(S_L, t, kvbuf.at[c - 1, mh], kvbuf.at[c - 1, mh],
                   here).wait_recv()

            @pl.when(my_y + 1 + t <= N_Y - 1)
            def _(t=t):
                c = my_y + 1 + t
                mk(S_XL, t, kvbuf.at[c - 1, mh], kvbuf.at[c - 1, mh],
                   twin_dev).start()

            if t + 1 < N_STEP:
                @pl.when(has_right & (my_y - (t + 1) >= 0))
                def _(t=t):
                    c = my_y - (t + 1)
                    mk(S_R, t + 1, kvbuf.at[c, mh], kvbuf.at[c, mh],
                       right_dev).start()

                @pl.when(has_left & (my_y + t + 1 <= N_Y - 1))
                def _(t=t):
                    c = my_y + t + 1
                    mk(S_L, t + 1, kvbuf.at[c - 1, mh], kvbuf.at[c - 1, mh],
                       left_dev).start()

            if t >= 1:
                @pl.when(my_y - 1 - (t - 1) >= 0)
                def _(t=t):
                    c = my_y - t
                    mk(S_XR, t - 1, kvbuf.at[c, oh], kvbuf.at[c, oh],
                       here).wait_recv()
                    consume_slot(c)

                @pl.when(my_y + t <= N_Y - 1)
                def _(t=t):
                    c = my_y + t
                    mk(S_XL, t - 1, kvbuf.at[c - 1, oh], kvbuf.at[c - 1, oh],
                       here).wait_recv()
                    consume_slot(c - 1)

        t_last = N_STEP - 1

        @pl.when(my_y - 1 - t_last >= 0)
        def _():
            c = my_y - 1 - t_last
            mk(S_XR, t_last, kvbuf.at[c, oh], kvbuf.at[c, oh],
               here).wait_recv()
            consume_slot(c)

        @pl.when(my_y + 1 + t_last <= N_Y - 1)
        def _():
            c = my_y + 1 + t_last
            mk(S_XL, t_last, kvbuf.at[c - 1, oh], kvbuf.at[c - 1, oh],
               here).wait_recv()
            consume_slot(c - 1)

        for t in range(N_STEP):
            @pl.when(has_right & (my_y - t >= 0))
            def _(t=t):
                c = my_y - t
                src = kv_ref.at[mh] if t == 0 else kvbuf.at[c, mh]
                mk(S_R, t, src, kvbuf.at[c, mh], right_dev).wait_send()

            @pl.when(has_left & (my_y + t <= N_Y - 1))
            def _(t=t):
                c = my_y + t
                src = kv_ref.at[mh] if t == 0 else kvbuf.at[c - 1, mh]
                mk(S_L, t, src, kvbuf.at[c - 1, mh], left_dev).wait_send()

            @pl.when(my_y - 1 - t >= 0)
            def _(t=t):
                c = my_y - 1 - t
                mk(S_XR, t, kvbuf.at[c, mh], kvbuf.at[c, mh],
                   twin_dev).wait_send()

            @pl.when(my_y + 1 + t <= N_Y - 1)
            def _(t=t):
                c = my_y + 1 + t
                mk(S_XL, t, kvbuf.at[c - 1, mh], kvbuf.at[c - 1, mh],
                   twin_dev).wait_send()

        def norm(i, carry):
            out_ref[i] = out_ref[i] / l_buf[i][None, :]
            return carry
        lax.fori_loop(0, bh, norm, 0)

    out = pl.pallas_call(
        body,
        out_shape=jax.ShapeDtypeStruct((bh, d, s), jnp.float32),
        in_specs=[pl.BlockSpec(memory_space=pltpu.VMEM)] * 2,
        out_specs=pl.BlockSpec(memory_space=pltpu.VMEM),
        scratch_shapes=[
            pltpu.VMEM((N_Y - 1, 2, 2, hh, d, s), jnp.float32),
            pltpu.VMEM((bh, s), jnp.float32),
            pltpu.VMEM((bh, s), jnp.float32),
            pltpu.SemaphoreType.DMA((4, N_STEP)),
            pltpu.SemaphoreType.DMA((4, N_STEP)),
        ],
        compiler_params=pltpu.CompilerParams(collective_id=0),
    )(Qt, KVt)

    return out.reshape(b, h, d, s).transpose(0, 3, 1, 2)
